# Initial kernel scaffold; baseline (speedup 1.0000x reference)
import jax
import jax.numpy as jnp
from jax.experimental import pallas as pl


def kernel(points_coords, centers_coords, points_features):
    raise NotImplementedError("write your pallas kernel here")


# speedup vs baseline: 295.4277x; 295.4277x over previous
import functools

import jax
import jax.numpy as jnp
from jax import lax
from jax.experimental import pallas as pl
from jax.experimental.pallas import tpu as pltpu
from jax.experimental.pallas import tpu_sc as plsc

RADIUS2 = 0.2 * 0.2
K = 32
D = 128
L = 16
NC = 2
NS = 16
NW = NC * NS


def _bf16_round(v):
    bits = plsc.bitcast(v, jnp.uint32)
    lsb = (bits >> 16) & jnp.uint32(1)
    bits = (bits + jnp.uint32(0x7FFF) + lsb) & jnp.uint32(0xFFFF0000)
    return plsc.bitcast(bits, jnp.float32)


def _sc_ball_query_gather(points_flat, centers_flat, table, B, N, M):
    rows_per_w = (B * M) // NW
    w_per_b = NW // B
    m_per_w = M // w_per_b

    mesh = plsc.VectorSubcoreMesh(core_axis_name="c", subcore_axis_name="s",
                                  num_cores=NC, num_subcores=NS)

    @functools.partial(
        pl.kernel,
        out_type=jax.ShapeDtypeStruct((B * M * K, D), jnp.float32),
        mesh=mesh,
        compiler_params=pltpu.CompilerParams(
            needs_layout_passes=False, use_tc_tiling_on_sc=False),
        scratch_types=[
            pltpu.VMEM((3 * N,), jnp.float32),
            pltpu.VMEM((3 * M,), jnp.float32),
            pltpu.VMEM((K,), jnp.int32),
            pltpu.VMEM((K,), jnp.int32),
            pltpu.VMEM((K, D), jnp.float32),
            pltpu.SemaphoreType.DMA,
        ],
    )
    def sc_kernel(points_hbm, centers_hbm, table_hbm, out_hbm,
                  pts_v, cen_v, slots_v, gidx_v, rows_v, sem):
        cid = lax.axis_index("c")
        sid = lax.axis_index("s")
        wid = sid * NC + cid
        b = wid // w_per_b
        wslot = wid % w_per_b
        pltpu.sync_copy(points_hbm.at[pl.ds(b * 3 * N, 3 * N)], pts_v)
        pltpu.sync_copy(centers_hbm.at[pl.ds(b * 3 * M, 3 * M)], cen_v)
        iota16 = lax.iota(jnp.int32, 16)

        def per_row(i, carry):
            m = i * w_per_b + wslot
            midx = jnp.full((L,), m, jnp.int32)
            cx = plsc.load_gather(cen_v, [midx])
            cy = plsc.load_gather(cen_v, [midx + M])
            cz = plsc.load_gather(cen_v, [midx + 2 * M])
            c2 = cx * cx + cy * cy + cz * cz
            cxb = _bf16_round(cx)
            cyb = _bf16_round(cy)
            czb = _bf16_round(cz)

            def cond(c):
                chunk, count = c
                return (chunk < N // L) & (count < K)

            def body(c):
                chunk, count = c
                base = chunk * L
                px = pts_v[pl.ds(base, L)]
                py = pts_v[pl.ds(base + N, L)]
                pz = pts_v[pl.ds(base + 2 * N, L)]
                p2 = px * px + py * py + pz * pz
                cp = (cxb * _bf16_round(px) + cyb * _bf16_round(py)
                      + czb * _bf16_round(pz))
                d2 = c2 + p2 - 2.0 * cp
                msk = d2 < RADIUS2
                mi = msk.astype(jnp.int32)
                csum = plsc.cumsum(mi)
                rank = count + csum - 1
                wmask = msk & (rank < K)
                rank_c = jnp.clip(rank, 0, K - 1)
                plsc.store_scatter(slots_v, [rank_c], base + iota16,
                                   mask=wmask)
                return (chunk + 1, count + jnp.sum(mi))

            _, count = lax.while_loop(cond, body,
                                      (jnp.int32(0), jnp.int32(0)))
            first = plsc.load_gather(slots_v, [jnp.full((L,), 0, jnp.int32)])
            first = jnp.where(count > 0, first, 0)
            for jj in range(K // L):
                valid = (iota16 + jj * L) < count
                cur = slots_v[pl.ds(jj * L, L)]
                filled = jnp.where(valid, cur, first)
                gidx_v[pl.ds(jj * L, L)] = filled + b * N
            pltpu.async_copy(table_hbm.at[gidx_v], rows_v, sem).wait()
            pltpu.sync_copy(rows_v, out_hbm.at[pl.ds((b * M + m) * K, K)])
            return carry

        lax.fori_loop(0, m_per_w, per_row, 0)

    return sc_kernel(points_flat, centers_flat, table)


def _tc_finalize(grouped, centers_coords):
    B, MK, _ = grouped.shape
    M = centers_coords.shape[-1]
    C_OUT = 67
    MB = 256
    BK = MB * K

    def body(g_ref, cen_ref, out_ref):
        x = g_ref[0]
        xt = x.T
        cen = cen_ref[0]
        cen_rep = jnp.broadcast_to(cen[:, :, None], (3, MB, K)).reshape(3, BK)
        out_ref[0, 0:3] = xt[0:3] - cen_rep
        out_ref[0, 3:C_OUT] = xt[3:C_OUT]

    return pl.pallas_call(
        body,
        grid=(B, M // MB),
        in_specs=[
            pl.BlockSpec((1, BK, D), lambda b, i: (b, i, 0)),
            pl.BlockSpec((1, 3, MB), lambda b, i: (b, 0, i)),
        ],
        out_specs=pl.BlockSpec((1, C_OUT, BK), lambda b, i: (b, 0, i)),
        out_shape=jax.ShapeDtypeStruct((B, C_OUT, MK), jnp.float32),
    )(grouped, centers_coords)


def kernel(points_coords, centers_coords, points_features):
    B, _, N = points_coords.shape
    M = centers_coords.shape[-1]
    table = jnp.concatenate([points_coords, points_features], axis=1)
    table = jnp.pad(table, ((0, 0), (0, D - table.shape[1]), (0, 0)))
    table = table.transpose(0, 2, 1).reshape(B * N, D)
    grouped = _sc_ball_query_gather(
        points_coords.reshape(B * 3 * N), centers_coords.reshape(B * 3 * M),
        table, B, N, M)
    grouped = grouped.reshape(B, M * K, D)
    out = _tc_finalize(grouped, centers_coords)
    return out.reshape(B, 67, M, K)


# speedup vs baseline: 344.2744x; 1.1653x over previous
import functools

import jax
import jax.numpy as jnp
from jax import lax
from jax.experimental import pallas as pl
from jax.experimental.pallas import tpu as pltpu
from jax.experimental.pallas import tpu_sc as plsc

RADIUS2 = 0.2 * 0.2
K = 32
D = 80
L = 16
NC = 2
NS = 16
NW = NC * NS
R = 2
NBUF = 4
AHEAD = 2


def _bf16_round(v):
    bits = plsc.bitcast(v, jnp.uint32)
    lsb = (bits >> 16) & jnp.uint32(1)
    bits = (bits + jnp.uint32(0x7FFF) + lsb) & jnp.uint32(0xFFFF0000)
    return plsc.bitcast(bits, jnp.float32)


def _sc_ball_query_gather(points_flat, centers_flat, table, B, N, M):
    w_per_b = NW // B
    m_per_w = M // w_per_b
    G = m_per_w // R
    RK = R * K

    mesh = plsc.VectorSubcoreMesh(core_axis_name="c", subcore_axis_name="s",
                                  num_cores=NC, num_subcores=NS)

    @functools.partial(
        pl.kernel,
        out_type=jax.ShapeDtypeStruct((B * M * K, D), jnp.float32),
        mesh=mesh,
        compiler_params=pltpu.CompilerParams(
            needs_layout_passes=False, use_tc_tiling_on_sc=False),
        scratch_types=[
            pltpu.VMEM((3 * N,), jnp.float32),
            pltpu.VMEM((3 * N,), jnp.float32),
            pltpu.VMEM((N,), jnp.float32),
            pltpu.VMEM((3 * M,), jnp.float32),
            pltpu.VMEM((K,), jnp.int32),
            pltpu.VMEM((m_per_w * K,), jnp.int32),
            [pltpu.VMEM((RK, D), jnp.float32) for _ in range(NBUF)],
            [pltpu.SemaphoreType.DMA for _ in range(NBUF)],
            [pltpu.SemaphoreType.DMA for _ in range(NBUF)],
        ],
    )
    def sc_kernel(points_hbm, centers_hbm, table_hbm, out_hbm,
                  pts_v, rnd_v, p2_v, cen_v, slots_v, gidx_v,
                  bufs, gsems, ssems):
        cid = lax.axis_index("c")
        sid = lax.axis_index("s")
        wid = sid * NC + cid
        b = wid // w_per_b
        wslot = wid % w_per_b
        m0 = wslot * m_per_w
        out0 = (b * M + m0) * K
        pltpu.sync_copy(points_hbm.at[pl.ds(b * 3 * N, 3 * N)], pts_v)
        pltpu.sync_copy(centers_hbm.at[pl.ds(b * 3 * M, 3 * M)], cen_v)
        iota16 = lax.iota(jnp.int32, 16)

        def prep(chunk, carry):
            base = chunk * L
            px = pts_v[pl.ds(base, L)]
            py = pts_v[pl.ds(base + N, L)]
            pz = pts_v[pl.ds(base + 2 * N, L)]
            p2_v[pl.ds(base, L)] = px * px + py * py + pz * pz
            rnd_v[pl.ds(base, L)] = _bf16_round(px)
            rnd_v[pl.ds(base + N, L)] = _bf16_round(py)
            rnd_v[pl.ds(base + 2 * N, L)] = _bf16_round(pz)
            return carry

        lax.fori_loop(0, N // L, prep, 0)

        def scan_row(i):
            m = m0 + i
            midx = jnp.full((L,), m, jnp.int32)
            cx = plsc.load_gather(cen_v, [midx])
            cy = plsc.load_gather(cen_v, [midx + M])
            cz = plsc.load_gather(cen_v, [midx + 2 * M])
            c2 = cx * cx + cy * cy + cz * cz
            cxb = _bf16_round(cx)
            cyb = _bf16_round(cy)
            czb = _bf16_round(cz)

            def cond(c):
                chunk, count = c
                return (chunk < N // L) & (count < K)

            def body(c):
                chunk, count = c
                base = chunk * L
                p2 = p2_v[pl.ds(base, L)]
                cp = (cxb * rnd_v[pl.ds(base, L)]
                      + cyb * rnd_v[pl.ds(base + N, L)]
                      + czb * rnd_v[pl.ds(base + 2 * N, L)])
                d2 = c2 + p2 - 2.0 * cp
                msk = d2 < RADIUS2
                mi = msk.astype(jnp.int32)
                csum = plsc.cumsum(mi)
                rank = count + csum - 1
                wmask = msk & (rank < K)
                rank_c = jnp.clip(rank, 0, K - 1)
                plsc.store_scatter(slots_v, [rank_c], base + iota16,
                                   mask=wmask)
                return (chunk + 1, count + jnp.sum(mi))

            _, count = lax.while_loop(cond, body,
                                      (jnp.int32(0), jnp.int32(0)))
            first = plsc.load_gather(slots_v, [jnp.full((L,), 0, jnp.int32)])
            first = jnp.where(count > 0, first, 0)
            for jj in range(K // L):
                valid = (iota16 + jj * L) < count
                cur = slots_v[pl.ds(jj * L, L)]
                filled = jnp.where(valid, cur, first)
                gidx_v[pl.ds(i * K + jj * L, L)] = filled + b * N

        def fire_gather(g, jbuf):
            idx = gidx_v.at[pl.ds(g * RK, RK)]
            pltpu.async_copy(table_hbm.at[idx], bufs[jbuf], gsems[jbuf])

        def fire_store(g, jbuf):
            pltpu.async_copy(bufs[jbuf],
                             out_hbm.at[pl.ds(out0 + g * RK, RK)],
                             ssems[jbuf])

        def wait_buf(sem, jbuf):
            pltpu.make_async_copy(out_hbm.at[pl.ds(0, RK)], bufs[jbuf],
                                  sem).wait()

        for g0 in range(AHEAD):
            scan_row(R * g0)
            scan_row(R * g0 + 1)
            fire_gather(g0, g0 % NBUF)

        def outer(o, carry):
            for j in range(NBUF):
                g = o * NBUF + j
                nf = g + AHEAD
                jn = (j + AHEAD) % NBUF

                @pl.when(nf < G)
                def _():
                    scan_row(R * nf)
                    scan_row(R * nf + 1)

                    @pl.when(nf >= NBUF)
                    def _():
                        wait_buf(ssems[jn], jn)

                    fire_gather(nf, jn)

                wait_buf(gsems[j], j)
                fire_store(g, j)
            return carry

        lax.fori_loop(0, G // NBUF, outer, 0)
        for j in range(NBUF):
            wait_buf(ssems[j], j)

    return sc_kernel(points_flat, centers_flat, table)


def _tc_finalize(grouped, centers_coords):
    B, MK, _ = grouped.shape
    M = centers_coords.shape[-1]
    C_OUT = 67
    MB = 256
    BK = MB * K

    def body(g_ref, cen_ref, out_ref):
        x = g_ref[0]
        xt = x.T
        cen = cen_ref[0]
        cen_rep = jnp.broadcast_to(cen[:, :, None], (3, MB, K)).reshape(3, BK)
        out_ref[0, 0:3] = xt[0:3] - cen_rep
        out_ref[0, 3:C_OUT] = xt[3:C_OUT]

    return pl.pallas_call(
        body,
        grid=(B, M // MB),
        in_specs=[
            pl.BlockSpec((1, BK, D), lambda b, i: (b, i, 0)),
            pl.BlockSpec((1, 3, MB), lambda b, i: (b, 0, i)),
        ],
        out_specs=pl.BlockSpec((1, C_OUT, BK), lambda b, i: (b, 0, i)),
        out_shape=jax.ShapeDtypeStruct((B, C_OUT, MK), jnp.float32),
    )(grouped, centers_coords)


def kernel(points_coords, centers_coords, points_features):
    B, _, N = points_coords.shape
    M = centers_coords.shape[-1]
    table = jnp.concatenate([points_coords, points_features], axis=1)
    table = jnp.pad(table, ((0, 0), (0, D - table.shape[1]), (0, 0)))
    table = table.transpose(0, 2, 1).reshape(B * N, D)
    grouped = _sc_ball_query_gather(
        points_coords.reshape(B * 3 * N), centers_coords.reshape(B * 3 * M),
        table, B, N, M)
    grouped = grouped.reshape(B, M * K, D)
    out = _tc_finalize(grouped, centers_coords)
    return out.reshape(B, 67, M, K)


# speedup vs baseline: 505.2918x; 1.4677x over previous
import functools

import jax
import jax.numpy as jnp
from jax import lax
from jax.experimental import pallas as pl
from jax.experimental.pallas import tpu as pltpu
from jax.experimental.pallas import tpu_sc as plsc

RADIUS2 = 0.2 * 0.2
K = 32
D = 80
L = 16
NC = 2
NS = 16
NW = NC * NS
R = 2
NBUF = 4
AHEAD = 2


def _bf16_round(v):
    bits = plsc.bitcast(v, jnp.uint32)
    lsb = (bits >> 16) & jnp.uint32(1)
    bits = (bits + jnp.uint32(0x7FFF) + lsb) & jnp.uint32(0xFFFF0000)
    return plsc.bitcast(bits, jnp.float32)


def _sc_ball_query_gather(points_flat, centers_flat, table, B, N, M):
    w_per_b = NW // B
    m_per_w = M // w_per_b
    G = m_per_w // R
    RK = R * K

    mesh = plsc.VectorSubcoreMesh(core_axis_name="c", subcore_axis_name="s",
                                  num_cores=NC, num_subcores=NS)

    @functools.partial(
        pl.kernel,
        out_type=jax.ShapeDtypeStruct((B * M * K, D), jnp.float32),
        mesh=mesh,
        compiler_params=pltpu.CompilerParams(
            needs_layout_passes=False, use_tc_tiling_on_sc=False),
        scratch_types=[
            pltpu.VMEM((3 * N,), jnp.float32),
            pltpu.VMEM((3 * N,), jnp.float32),
            pltpu.VMEM((N,), jnp.float32),
            pltpu.VMEM((3 * M,), jnp.float32),
            pltpu.VMEM((K,), jnp.int32),
            pltpu.VMEM((m_per_w * K,), jnp.int32),
            [pltpu.VMEM((RK, D), jnp.float32) for _ in range(NBUF)],
            [pltpu.SemaphoreType.DMA for _ in range(NBUF)],
            [pltpu.SemaphoreType.DMA for _ in range(NBUF)],
        ],
    )
    def sc_kernel(points_hbm, centers_hbm, table_hbm, out_hbm,
                  pts_v, rnd_v, p2_v, cen_v, slots_v, gidx_v,
                  bufs, gsems, ssems):
        cid = lax.axis_index("c")
        sid = lax.axis_index("s")
        wid = sid * NC + cid
        b = wid // w_per_b
        wslot = wid % w_per_b
        m0 = wslot * m_per_w
        out0 = (b * M + m0) * K
        pltpu.sync_copy(points_hbm.at[pl.ds(b * 3 * N, 3 * N)], pts_v)
        pltpu.sync_copy(centers_hbm.at[pl.ds(b * 3 * M, 3 * M)], cen_v)
        iota16 = lax.iota(jnp.int32, 16)

        def prep(chunk, carry):
            base = chunk * L
            px = pts_v[pl.ds(base, L)]
            py = pts_v[pl.ds(base + N, L)]
            pz = pts_v[pl.ds(base + 2 * N, L)]
            p2_v[pl.ds(base, L)] = px * px + py * py + pz * pz
            rnd_v[pl.ds(base, L)] = _bf16_round(px)
            rnd_v[pl.ds(base + N, L)] = _bf16_round(py)
            rnd_v[pl.ds(base + 2 * N, L)] = _bf16_round(pz)
            return carry

        lax.fori_loop(0, N // L, prep, 0)

        def scan_row(i):
            m = m0 + i
            midx = jnp.full((L,), m, jnp.int32)
            cx = plsc.load_gather(cen_v, [midx])
            cy = plsc.load_gather(cen_v, [midx + M])
            cz = plsc.load_gather(cen_v, [midx + 2 * M])
            c2 = cx * cx + cy * cy + cz * cz
            cxb = _bf16_round(cx)
            cyb = _bf16_round(cy)
            czb = _bf16_round(cz)

            def mask_at(base):
                p2 = p2_v[pl.ds(base, L)]
                cp = (cxb * rnd_v[pl.ds(base, L)]
                      + cyb * rnd_v[pl.ds(base + N, L)]
                      + czb * rnd_v[pl.ds(base + 2 * N, L)])
                d2 = c2 + p2 - 2.0 * cp
                return d2 < RADIUS2

            def cond(c):
                chunk, count_v = c
                return (chunk < N // L) & (count_v[0] < K)

            def body(c):
                chunk, count_v = c
                base = chunk * L
                m1 = mask_at(base)
                m2 = mask_at(base + L)
                pc1 = plsc.all_reduce_population_count(m1)
                pc2 = plsc.all_reduce_population_count(m2)
                rank1 = count_v + plsc.cumsum(m1.astype(jnp.int32)) - 1
                rank2 = (count_v + pc1
                         + plsc.cumsum(m2.astype(jnp.int32)) - 1)
                plsc.store_scatter(slots_v, [jnp.clip(rank1, 0, K - 1)],
                                   base + iota16, mask=m1 & (rank1 < K))
                plsc.store_scatter(slots_v, [jnp.clip(rank2, 0, K - 1)],
                                   base + L + iota16, mask=m2 & (rank2 < K))
                return (chunk + 2, count_v + (pc1 + pc2))

            _, count_v = lax.while_loop(
                cond, body, (jnp.int32(0), jnp.zeros((L,), jnp.int32)))
            first = plsc.load_gather(slots_v, [jnp.full((L,), 0, jnp.int32)])
            first = jnp.where(count_v > 0, first, 0)
            for jj in range(K // L):
                valid = (iota16 + jj * L) < count_v
                cur = slots_v[pl.ds(jj * L, L)]
                filled = jnp.where(valid, cur, first)
                gidx_v[pl.ds(i * K + jj * L, L)] = filled + b * N

        def fire_gather(g, jbuf):
            idx = gidx_v.at[pl.ds(g * RK, RK)]
            pltpu.async_copy(table_hbm.at[idx], bufs[jbuf], gsems[jbuf])

        def fire_store(g, jbuf):
            pltpu.async_copy(bufs[jbuf],
                             out_hbm.at[pl.ds(out0 + g * RK, RK)],
                             ssems[jbuf])

        def wait_buf(sem, jbuf):
            pltpu.make_async_copy(out_hbm.at[pl.ds(0, RK)], bufs[jbuf],
                                  sem).wait()

        for g0 in range(AHEAD):
            scan_row(R * g0)
            scan_row(R * g0 + 1)
            fire_gather(g0, g0 % NBUF)

        def outer(o, carry):
            for j in range(NBUF):
                g = o * NBUF + j
                nf = g + AHEAD
                jn = (j + AHEAD) % NBUF

                @pl.when(nf < G)
                def _():
                    scan_row(R * nf)
                    scan_row(R * nf + 1)

                    @pl.when(nf >= NBUF)
                    def _():
                        wait_buf(ssems[jn], jn)

                    fire_gather(nf, jn)

                wait_buf(gsems[j], j)
                fire_store(g, j)
            return carry

        lax.fori_loop(0, G // NBUF, outer, 0)
        for j in range(NBUF):
            wait_buf(ssems[j], j)

    return sc_kernel(points_flat, centers_flat, table)


def _tc_finalize(grouped, centers_coords):
    B, MK, _ = grouped.shape
    M = centers_coords.shape[-1]
    C_OUT = 67
    MB = 256
    BK = MB * K

    def body(g_ref, cen_ref, out_ref):
        x = g_ref[0]
        xt = x.T
        cen = cen_ref[0]
        cen_rep = jnp.broadcast_to(cen[:, :, None], (3, MB, K)).reshape(3, BK)
        out_ref[0, 0:3] = xt[0:3] - cen_rep
        out_ref[0, 3:C_OUT] = xt[3:C_OUT]

    return pl.pallas_call(
        body,
        grid=(B, M // MB),
        in_specs=[
            pl.BlockSpec((1, BK, D), lambda b, i: (b, i, 0)),
            pl.BlockSpec((1, 3, MB), lambda b, i: (b, 0, i)),
        ],
        out_specs=pl.BlockSpec((1, C_OUT, BK), lambda b, i: (b, 0, i)),
        out_shape=jax.ShapeDtypeStruct((B, C_OUT, MK), jnp.float32),
    )(grouped, centers_coords)


def kernel(points_coords, centers_coords, points_features):
    B, _, N = points_coords.shape
    M = centers_coords.shape[-1]
    table = jnp.concatenate([points_coords, points_features], axis=1)
    table = jnp.pad(table, ((0, 0), (0, D - table.shape[1]), (0, 0)))
    table = table.transpose(0, 2, 1).reshape(B * N, D)
    grouped = _sc_ball_query_gather(
        points_coords.reshape(B * 3 * N), centers_coords.reshape(B * 3 * M),
        table, B, N, M)
    grouped = grouped.reshape(B, M * K, D)
    out = _tc_finalize(grouped, centers_coords)
    return out.reshape(B, 67, M, K)


# speedup vs baseline: 570.9232x; 1.1299x over previous
import functools

import jax
import jax.numpy as jnp
from jax import lax
from jax.experimental import pallas as pl
from jax.experimental.pallas import tpu as pltpu
from jax.experimental.pallas import tpu_sc as plsc

RADIUS2 = 0.2 * 0.2
K = 32
D = 128
L = 16
NC = 2
NS = 16
NW = NC * NS
R = 2
NBUF = 4
AHEAD = 2


def _bf16_round(v):
    bits = plsc.bitcast(v, jnp.uint32)
    lsb = (bits >> 16) & jnp.uint32(1)
    bits = (bits + jnp.uint32(0x7FFF) + lsb) & jnp.uint32(0xFFFF0000)
    return plsc.bitcast(bits, jnp.float32)


def _sc_ball_query_gather(points_flat, centers_flat, table, B, N, M):
    w_per_b = NW // B
    m_per_w = M // w_per_b
    G = m_per_w // R
    RK = R * K

    mesh = plsc.VectorSubcoreMesh(core_axis_name="c", subcore_axis_name="s",
                                  num_cores=NC, num_subcores=NS)

    @functools.partial(
        pl.kernel,
        out_type=jax.ShapeDtypeStruct((B * M * K, D), jnp.float32),
        mesh=mesh,
        compiler_params=pltpu.CompilerParams(
            needs_layout_passes=False, use_tc_tiling_on_sc=True),
        scratch_types=[
            pltpu.VMEM((3 * N,), jnp.float32),
            pltpu.VMEM((3 * N,), jnp.float32),
            pltpu.VMEM((N,), jnp.float32),
            pltpu.VMEM((3 * M,), jnp.float32),
            pltpu.VMEM((K,), jnp.int32),
            pltpu.VMEM((m_per_w * K,), jnp.int32),
            [pltpu.VMEM((RK, D), jnp.float32) for _ in range(NBUF)],
            [pltpu.SemaphoreType.DMA for _ in range(NBUF)],
            [pltpu.SemaphoreType.DMA for _ in range(NBUF)],
        ],
    )
    def sc_kernel(points_hbm, centers_hbm, table_hbm, out_hbm,
                  pts_v, rnd_v, p2_v, cen_v, slots_v, gidx_v,
                  bufs, gsems, ssems):
        cid = lax.axis_index("c")
        sid = lax.axis_index("s")
        wid = sid * NC + cid
        b = wid // w_per_b
        wslot = wid % w_per_b
        m0 = wslot * m_per_w
        out0 = (b * M + m0) * K
        pltpu.sync_copy(points_hbm.at[pl.ds(b * 3 * N, 3 * N)], pts_v)
        pltpu.sync_copy(centers_hbm.at[pl.ds(b * 3 * M, 3 * M)], cen_v)
        iota16 = lax.iota(jnp.int32, 16)

        def prep(chunk, carry):
            base = chunk * L
            px = pts_v[pl.ds(base, L)]
            py = pts_v[pl.ds(base + N, L)]
            pz = pts_v[pl.ds(base + 2 * N, L)]
            p2_v[pl.ds(base, L)] = px * px + py * py + pz * pz
            rnd_v[pl.ds(base, L)] = _bf16_round(px)
            rnd_v[pl.ds(base + N, L)] = _bf16_round(py)
            rnd_v[pl.ds(base + 2 * N, L)] = _bf16_round(pz)
            return carry

        lax.fori_loop(0, N // L, prep, 0)

        def scan_row(i):
            m = m0 + i
            midx = jnp.full((L,), m, jnp.int32)
            cx = plsc.load_gather(cen_v, [midx])
            cy = plsc.load_gather(cen_v, [midx + M])
            cz = plsc.load_gather(cen_v, [midx + 2 * M])
            c2 = cx * cx + cy * cy + cz * cz
            cxb = _bf16_round(cx)
            cyb = _bf16_round(cy)
            czb = _bf16_round(cz)

            def mask_at(base):
                p2 = p2_v[pl.ds(base, L)]
                cp = (cxb * rnd_v[pl.ds(base, L)]
                      + cyb * rnd_v[pl.ds(base + N, L)]
                      + czb * rnd_v[pl.ds(base + 2 * N, L)])
                d2 = c2 + p2 - 2.0 * cp
                return d2 < RADIUS2

            def cond(c):
                chunk, count_v = c
                return (chunk < N // L) & (count_v[0] < K)

            def body(c):
                chunk, count_v = c
                base = chunk * L
                m1 = mask_at(base)
                m2 = mask_at(base + L)
                pc1 = plsc.all_reduce_population_count(m1)
                pc2 = plsc.all_reduce_population_count(m2)
                rank1 = count_v + plsc.cumsum(m1.astype(jnp.int32)) - 1
                rank2 = (count_v + pc1
                         + plsc.cumsum(m2.astype(jnp.int32)) - 1)
                plsc.store_scatter(slots_v, [jnp.clip(rank1, 0, K - 1)],
                                   base + iota16, mask=m1 & (rank1 < K))
                plsc.store_scatter(slots_v, [jnp.clip(rank2, 0, K - 1)],
                                   base + L + iota16, mask=m2 & (rank2 < K))
                return (chunk + 2, count_v + (pc1 + pc2))

            _, count_v = lax.while_loop(
                cond, body, (jnp.int32(0), jnp.zeros((L,), jnp.int32)))
            first = plsc.load_gather(slots_v, [jnp.full((L,), 0, jnp.int32)])
            first = jnp.where(count_v > 0, first, 0)
            for jj in range(K // L):
                valid = (iota16 + jj * L) < count_v
                cur = slots_v[pl.ds(jj * L, L)]
                filled = jnp.where(valid, cur, first)
                gidx_v[pl.ds(i * K + jj * L, L)] = filled + b * N

        def fire_gather(g, jbuf):
            idx = gidx_v.at[pl.ds(g * RK, RK)]
            pltpu.async_copy(table_hbm.at[idx], bufs[jbuf], gsems[jbuf])

        def fire_store(g, jbuf):
            pltpu.async_copy(bufs[jbuf],
                             out_hbm.at[pl.ds(out0 + g * RK, RK)],
                             ssems[jbuf])

        def wait_buf(sem, jbuf):
            pltpu.make_async_copy(out_hbm.at[pl.ds(0, RK)], bufs[jbuf],
                                  sem).wait()

        for g0 in range(AHEAD):
            scan_row(R * g0)
            scan_row(R * g0 + 1)
            fire_gather(g0, g0 % NBUF)

        def outer(o, carry):
            for j in range(NBUF):
                g = o * NBUF + j
                nf = g + AHEAD
                jn = (j + AHEAD) % NBUF

                @pl.when(nf < G)
                def _():
                    scan_row(R * nf)
                    scan_row(R * nf + 1)

                    @pl.when(nf >= NBUF)
                    def _():
                        wait_buf(ssems[jn], jn)

                    fire_gather(nf, jn)

                wait_buf(gsems[j], j)
                fire_store(g, j)
            return carry

        lax.fori_loop(0, G // NBUF, outer, 0)
        for j in range(NBUF):
            wait_buf(ssems[j], j)

    return sc_kernel(points_flat, centers_flat, table)


def _tc_finalize(grouped, centers_coords):
    BMK = grouped.shape[0]
    B, _, M = centers_coords.shape
    C_OUT = 67
    MB = 256
    BK = MB * K
    NBLK = M // MB

    def body(g_ref, cen_ref, out_ref):
        x = g_ref[...]
        xt = x.T
        cen = cen_ref[0]
        out_ref[0, 0:3] = (xt[0:3].reshape(3, MB, K)
                           - cen[:, :, None])
        out_ref[0, 3:C_OUT] = xt[3:C_OUT].reshape(C_OUT - 3, MB, K)

    return pl.pallas_call(
        body,
        grid=(B, NBLK),
        in_specs=[
            pl.BlockSpec((BK, D), lambda b, i: (b * NBLK + i, 0)),
            pl.BlockSpec((1, 3, MB), lambda b, i: (b, 0, i)),
        ],
        out_specs=pl.BlockSpec((1, C_OUT, MB, K), lambda b, i: (b, 0, i, 0)),
        out_shape=jax.ShapeDtypeStruct((B, C_OUT, M, K), jnp.float32),
    )(grouped, centers_coords)


def kernel(points_coords, centers_coords, points_features):
    B, _, N = points_coords.shape
    M = centers_coords.shape[-1]
    table = jnp.concatenate([points_coords, points_features], axis=1)
    table = jnp.pad(table, ((0, 0), (0, D - table.shape[1]), (0, 0)))
    table = table.transpose(0, 2, 1).reshape(B * N, D)
    grouped = _sc_ball_query_gather(
        points_coords.reshape(B * 3 * N), centers_coords.reshape(B * 3 * M),
        table, B, N, M)
    return _tc_finalize(grouped, centers_coords)
